# Initial kernel scaffold; baseline (speedup 1.0000x reference)
import jax
import jax.numpy as jnp
from jax.experimental import pallas as pl


def kernel(x, edge_index, W1, b1, W2, b2):
    raise NotImplementedError("write your pallas kernel here")


# speedup vs baseline: 28.4505x; 28.4505x over previous
import functools

import jax
import jax.numpy as jnp
from jax import lax
from jax.experimental import pallas as pl
from jax.experimental.pallas import tpu as pltpu
from jax.experimental.pallas import tpu_sc as plsc

NC = 2
NS = 16
NW = NC * NS
LANES = 16
CHUNK = 128
_BLK = 1024


def _sc_mesh():
    return plsc.VectorSubcoreMesh(
        core_axis_name="c", subcore_axis_name="s",
        num_cores=NC, num_subcores=NS)


def _make_deg_kernel(R, RW):
    rpt = R // NS

    nfull, rem = divmod(rpt, CHUNK)
    sizes = [CHUNK] * nfull + ([rem] if rem else [])

    @functools.partial(
        pl.kernel,
        out_type=jax.ShapeDtypeStruct((NC * R,), jnp.float32),
        mesh=_sc_mesh(),
        scratch_types=[
            pltpu.VMEM((RW, CHUNK), jnp.int32),
            pltpu.VMEM((CHUNK,), jnp.float32),
            pltpu.VMEM((CHUNK,), jnp.float32),
            pltpu.VMEM_SHARED((R,), jnp.float32),
        ],
    )
    def deg_kernel(dst_hbm, zdeg_hbm, out_hbm, idx_v, ones_v, stage_v, hist_s):
        cid = lax.axis_index("c")
        sid = lax.axis_index("s")
        wid = cid * NS + sid
        for k in range(CHUNK // LANES):
            ones_v[pl.ds(k * LANES, LANES)] = jnp.ones((LANES,), jnp.float32)
        pltpu.sync_copy(zdeg_hbm.at[pl.ds(0, CHUNK)], stage_v)
        off = 0
        for sz in sizes:
            pltpu.sync_copy(stage_v.at[pl.ds(0, sz)],
                            hist_s.at[pl.ds(sid * rpt + off, sz)])
            off += sz
        pltpu.sync_copy(dst_hbm.at[pl.ds(wid * RW, RW)], idx_v)
        plsc.subcore_barrier()

        def body(j, carry):
            pltpu.sync_copy(ones_v, hist_s.at[idx_v.at[j]], add=True)
            return carry

        lax.fori_loop(0, RW, body, 0)
        plsc.subcore_barrier()
        off = 0
        for sz in sizes:
            pltpu.sync_copy(hist_s.at[pl.ds(sid * rpt + off, sz)],
                            stage_v.at[pl.ds(0, sz)])
            pltpu.sync_copy(stage_v.at[pl.ds(0, sz)],
                            out_hbm.at[pl.ds(cid * R + sid * rpt + off, sz)])
            off += sz

    return deg_kernel


def _make_agg_kernel(R, RW, F):
    rpt = R // NS
    nfull, rem = divmod(rpt, CHUNK)
    sizes = [CHUNK] * nfull + ([rem] if rem else [])
    assert RW % 16 == 0 or RW % 8 == 0
    HALF = RW // 2

    @functools.partial(
        pl.kernel,
        out_type=jax.ShapeDtypeStruct((NC * R, F), jnp.float32),
        mesh=_sc_mesh(),
        scratch_types=[
            pltpu.VMEM((HALF, CHUNK), jnp.int32),
            pltpu.VMEM((HALF, CHUNK), jnp.int32),
            pltpu.VMEM((CHUNK, F), jnp.float32),
            pltpu.VMEM((CHUNK, F), jnp.float32),
            pltpu.VMEM_SHARED((R, F), jnp.float32),
            pltpu.SemaphoreType.DMA,
            pltpu.SemaphoreType.DMA,
        ],
    )
    def agg_kernel(xs_hbm, src_hbm, dst_hbm, zacc_hbm, out_hbm,
                   src_v, dst_v, buf0, buf1, acc_s, sem0, sem1):
        cid = lax.axis_index("c")
        sid = lax.axis_index("s")
        wid = cid * NS + sid
        pltpu.sync_copy(zacc_hbm.at[pl.ds(0, CHUNK)], buf0)
        off = 0
        for sz in sizes:
            pltpu.sync_copy(buf0.at[pl.ds(0, sz)],
                            acc_s.at[pl.ds(sid * rpt + off, sz)])
            off += sz
        plsc.subcore_barrier()

        def body(jj, carry):
            j0 = 2 * jj
            j1 = j0 + 1
            c0 = pltpu.async_copy(xs_hbm.at[src_v.at[j0]], buf0, sem0)
            c1 = pltpu.async_copy(xs_hbm.at[src_v.at[j1]], buf1, sem1)
            c0.wait()
            pltpu.sync_copy(buf0, acc_s.at[dst_v.at[j0]], add=True)
            c1.wait()
            pltpu.sync_copy(buf1, acc_s.at[dst_v.at[j1]], add=True)
            return carry

        for half in range(2):
            pltpu.sync_copy(src_hbm.at[pl.ds(wid * RW + half * HALF, HALF)],
                            src_v)
            pltpu.sync_copy(dst_hbm.at[pl.ds(wid * RW + half * HALF, HALF)],
                            dst_v)
            lax.fori_loop(0, HALF // 2, body, 0)
        plsc.subcore_barrier()
        off = 0
        for sz in sizes:
            pltpu.sync_copy(acc_s.at[pl.ds(sid * rpt + off, sz)],
                            buf0.at[pl.ds(0, sz)])
            pltpu.sync_copy(buf0.at[pl.ds(0, sz)],
                            out_hbm.at[pl.ds(cid * R + sid * rpt + off, sz), :])
            off += sz

    return agg_kernel


def _tc_prep(degp, x):
    N, F = x.shape

    def body(degp_ref, x_ref, dinvb_ref, xs_ref):
        deg = degp_ref[0, :] + degp_ref[1, :] + 1.0
        dinv = lax.rsqrt(deg)
        d2 = jnp.broadcast_to(dinv[:, None], x_ref.shape)
        dinvb_ref[...] = d2
        xs_ref[...] = x_ref[...] * d2

    return pl.pallas_call(
        body,
        grid=(pl.cdiv(N, _BLK),),
        in_specs=[pl.BlockSpec((NC, _BLK), lambda i: (0, i)),
                  pl.BlockSpec((_BLK, F), lambda i: (i, 0))],
        out_specs=[pl.BlockSpec((_BLK, F), lambda i: (i, 0)),
                   pl.BlockSpec((_BLK, F), lambda i: (i, 0))],
        out_shape=[jax.ShapeDtypeStruct((N, F), jnp.float32),
                   jax.ShapeDtypeStruct((N, F), jnp.float32)],
    )(degp, x)


def _tc_mid(p, xs, dinvb, W1, b1, W2):
    N, F = xs.shape
    H = W1.shape[1]

    def body(p_ref, xs_ref, dinvb_ref, w1_ref, b1_ref, w2_ref, out_ref):
        dinv = dinvb_ref[...]
        u = dinv * (p_ref[0] + p_ref[1] + xs_ref[...])
        h = jnp.dot(u, w1_ref[...], preferred_element_type=jnp.float32)
        h = jnp.maximum(h + b1_ref[...], 0.0)
        t = jnp.dot(h, w2_ref[...], preferred_element_type=jnp.float32)
        out_ref[...] = dinv * t

    return pl.pallas_call(
        body,
        grid=(pl.cdiv(N, _BLK),),
        in_specs=[pl.BlockSpec((NC, _BLK, F), lambda i: (0, i, 0)),
                  pl.BlockSpec((_BLK, F), lambda i: (i, 0)),
                  pl.BlockSpec((_BLK, F), lambda i: (i, 0)),
                  pl.BlockSpec((F, H), lambda i: (0, 0)),
                  pl.BlockSpec((1, H), lambda i: (0, 0)),
                  pl.BlockSpec((H, F), lambda i: (0, 0))],
        out_specs=pl.BlockSpec((_BLK, F), lambda i: (i, 0)),
        out_shape=jax.ShapeDtypeStruct((N, F), jnp.float32),
    )(p, xs, dinvb, W1, b1.reshape(1, H), W2)


def _tc_fin(q, ts, dinvb, b2):
    N, F = ts.shape

    def body(q_ref, ts_ref, dinvb_ref, b2_ref, out_ref):
        out_ref[...] = (dinvb_ref[...] * (q_ref[0] + q_ref[1] + ts_ref[...])
                        + b2_ref[...])

    return pl.pallas_call(
        body,
        grid=(pl.cdiv(N, _BLK),),
        in_specs=[pl.BlockSpec((NC, _BLK, F), lambda i: (0, i, 0)),
                  pl.BlockSpec((_BLK, F), lambda i: (i, 0)),
                  pl.BlockSpec((_BLK, F), lambda i: (i, 0)),
                  pl.BlockSpec((1, F), lambda i: (0, 0))],
        out_specs=pl.BlockSpec((_BLK, F), lambda i: (i, 0)),
        out_shape=jax.ShapeDtypeStruct((N, F), jnp.float32),
    )(q, ts, dinvb, b2.reshape(1, F))


def kernel(x, edge_index, W1, b1, W2, b2):
    N, F = x.shape
    E = edge_index.shape[1]
    R = ((N + 127) // 128) * 128
    EP = pl.cdiv(E, CHUNK * NW * 8) * (CHUNK * NW * 8)
    RW = EP // (CHUNK * NW)
    padn = EP - E

    ei = edge_index.astype(jnp.int32)
    pad_src = (jnp.arange(padn, dtype=jnp.int32) * 97) % N
    pad_dst = N + (jnp.arange(padn, dtype=jnp.int32) % (R - N))
    src2d = jnp.concatenate([ei[0], pad_src]).reshape(EP // CHUNK, CHUNK)
    dst2d = jnp.concatenate([ei[1], pad_dst]).reshape(EP // CHUNK, CHUNK)
    zdeg = jnp.zeros((R,), jnp.float32)
    zacc = jnp.zeros((R, F), jnp.float32)

    degp = _make_deg_kernel(R, RW)(dst2d, zdeg).reshape(NC, R)
    dinvb, xs = _tc_prep(degp, x)
    agg = _make_agg_kernel(R, RW, F)
    p = agg(xs, src2d, dst2d, zacc).reshape(NC, R, F)
    ts = _tc_mid(p, xs, dinvb, W1, b1, W2)
    q = agg(ts, src2d, dst2d, zacc).reshape(NC, R, F)
    return _tc_fin(q, ts, dinvb, b2)


# speedup vs baseline: 29.4969x; 1.0368x over previous
import functools

import jax
import jax.numpy as jnp
from jax import lax
from jax.experimental import pallas as pl
from jax.experimental.pallas import tpu as pltpu
from jax.experimental.pallas import tpu_sc as plsc

NC = 2
NS = 16
NW = NC * NS
LANES = 16
CHUNK = 128
_BLK = 1024


def _sc_mesh():
    return plsc.VectorSubcoreMesh(
        core_axis_name="c", subcore_axis_name="s",
        num_cores=NC, num_subcores=NS)


def _make_deg_kernel(R, RW):
    rpt = R // NS

    nfull, rem = divmod(rpt, CHUNK)
    sizes = [CHUNK] * nfull + ([rem] if rem else [])

    @functools.partial(
        pl.kernel,
        out_type=jax.ShapeDtypeStruct((NC * R,), jnp.float32),
        mesh=_sc_mesh(),
        scratch_types=[
            pltpu.VMEM((RW, CHUNK), jnp.int32),
            pltpu.VMEM((CHUNK,), jnp.float32),
            pltpu.VMEM((CHUNK,), jnp.float32),
            pltpu.VMEM_SHARED((R,), jnp.float32),
        ],
    )
    def deg_kernel(dst_hbm, zdeg_hbm, out_hbm, idx_v, ones_v, stage_v, hist_s):
        cid = lax.axis_index("c")
        sid = lax.axis_index("s")
        wid = cid * NS + sid
        for k in range(CHUNK // LANES):
            ones_v[pl.ds(k * LANES, LANES)] = jnp.ones((LANES,), jnp.float32)
        pltpu.sync_copy(zdeg_hbm.at[pl.ds(0, CHUNK)], stage_v)
        off = 0
        for sz in sizes:
            pltpu.sync_copy(stage_v.at[pl.ds(0, sz)],
                            hist_s.at[pl.ds(sid * rpt + off, sz)])
            off += sz
        pltpu.sync_copy(dst_hbm.at[pl.ds(wid * RW, RW)], idx_v)
        plsc.subcore_barrier()

        def body(j, carry):
            pltpu.sync_copy(ones_v, hist_s.at[idx_v.at[j]], add=True)
            return carry

        lax.fori_loop(0, RW, body, 0)
        plsc.subcore_barrier()
        off = 0
        for sz in sizes:
            pltpu.sync_copy(hist_s.at[pl.ds(sid * rpt + off, sz)],
                            stage_v.at[pl.ds(0, sz)])
            pltpu.sync_copy(stage_v.at[pl.ds(0, sz)],
                            out_hbm.at[pl.ds(cid * R + sid * rpt + off, sz)])
            off += sz

    return deg_kernel


def _make_agg_kernel(R, RW, F):
    rpt = R // NS
    nfull, rem = divmod(rpt, CHUNK)
    sizes = [CHUNK] * nfull + ([rem] if rem else [])
    assert RW % 16 == 0 or RW % 8 == 0
    HALF = RW // 2

    @functools.partial(
        pl.kernel,
        out_type=jax.ShapeDtypeStruct((NC * R, F), jnp.float32),
        mesh=_sc_mesh(),
        scratch_types=[
            pltpu.VMEM((HALF, CHUNK), jnp.int32),
            pltpu.VMEM((HALF, CHUNK), jnp.int32),
            pltpu.VMEM((CHUNK, F), jnp.float32),
            pltpu.VMEM((CHUNK, F), jnp.float32),
            pltpu.VMEM_SHARED((R, F), jnp.float32),
            pltpu.SemaphoreType.DMA,
            pltpu.SemaphoreType.DMA,
            pltpu.SemaphoreType.DMA,
            pltpu.SemaphoreType.DMA,
        ],
    )
    def agg_kernel(xs_hbm, src_hbm, dst_hbm, zacc_hbm, out_hbm,
                   src_v, dst_v, buf0, buf1, acc_s, gsem0, gsem1, ssem0, ssem1):
        cid = lax.axis_index("c")
        sid = lax.axis_index("s")
        wid = cid * NS + sid
        pltpu.sync_copy(zacc_hbm.at[pl.ds(0, CHUNK)], buf0)
        hs = []
        off = 0
        for i, sz in enumerate(sizes):
            hs.append(pltpu.async_copy(
                buf0.at[pl.ds(0, sz)],
                acc_s.at[pl.ds(sid * rpt + off, sz)],
                ssem0 if i % 2 == 0 else ssem1))
            off += sz
        for h in hs:
            h.wait()
        plsc.subcore_barrier()

        def g_wait():
            pltpu.make_async_copy(xs_hbm.at[src_v.at[0]], buf0, gsem0).wait()

        def g_wait1():
            pltpu.make_async_copy(xs_hbm.at[src_v.at[0]], buf1, gsem1).wait()

        def s_wait():
            pltpu.make_async_copy(buf0, acc_s.at[dst_v.at[0]], ssem0).wait()

        def s_wait1():
            pltpu.make_async_copy(buf1, acc_s.at[dst_v.at[0]], ssem1).wait()

        npair = HALF // 2

        def body(jj, carry):
            j0 = 2 * jj
            j1 = j0 + 1
            g_wait()
            pltpu.async_copy(buf0, acc_s.at[dst_v.at[j0]], ssem0, add=True)
            g_wait1()
            pltpu.async_copy(buf1, acc_s.at[dst_v.at[j1]], ssem1, add=True)
            s_wait()

            @pl.when(jj < npair - 1)
            def _():
                pltpu.async_copy(xs_hbm.at[src_v.at[j0 + 2]], buf0, gsem0)

            s_wait1()

            @pl.when(jj < npair - 1)
            def _():
                pltpu.async_copy(xs_hbm.at[src_v.at[j1 + 2]], buf1, gsem1)

            return carry

        for half in range(2):
            pltpu.sync_copy(src_hbm.at[pl.ds(wid * RW + half * HALF, HALF)],
                            src_v)
            pltpu.sync_copy(dst_hbm.at[pl.ds(wid * RW + half * HALF, HALF)],
                            dst_v)
            pltpu.async_copy(xs_hbm.at[src_v.at[0]], buf0, gsem0)
            pltpu.async_copy(xs_hbm.at[src_v.at[1]], buf1, gsem1)
            lax.fori_loop(0, npair, body, 0)
        plsc.subcore_barrier()
        off = 0
        whs = [None, None]
        for i, sz in enumerate(sizes):
            b = buf0 if i % 2 == 0 else buf1
            sem = ssem0 if i % 2 == 0 else ssem1
            if whs[i % 2] is not None:
                whs[i % 2].wait()
            pltpu.sync_copy(acc_s.at[pl.ds(sid * rpt + off, sz)],
                            b.at[pl.ds(0, sz)])
            whs[i % 2] = pltpu.async_copy(
                b.at[pl.ds(0, sz)],
                out_hbm.at[pl.ds(cid * R + sid * rpt + off, sz), :], sem)
            off += sz
        for h in whs:
            if h is not None:
                h.wait()

    return agg_kernel


def _tc_prep(degp, x):
    N, F = x.shape

    def body(degp_ref, x_ref, dinvb_ref, xs_ref):
        deg = degp_ref[0, :] + degp_ref[1, :] + 1.0
        dinv = lax.rsqrt(deg)
        d2 = jnp.broadcast_to(dinv[:, None], x_ref.shape)
        dinvb_ref[...] = d2
        xs_ref[...] = x_ref[...] * d2

    return pl.pallas_call(
        body,
        grid=(pl.cdiv(N, _BLK),),
        in_specs=[pl.BlockSpec((NC, _BLK), lambda i: (0, i)),
                  pl.BlockSpec((_BLK, F), lambda i: (i, 0))],
        out_specs=[pl.BlockSpec((_BLK, F), lambda i: (i, 0)),
                   pl.BlockSpec((_BLK, F), lambda i: (i, 0))],
        out_shape=[jax.ShapeDtypeStruct((N, F), jnp.float32),
                   jax.ShapeDtypeStruct((N, F), jnp.float32)],
    )(degp, x)


def _tc_mid(p, xs, dinvb, W1, b1, W2):
    N, F = xs.shape
    H = W1.shape[1]

    def body(p_ref, xs_ref, dinvb_ref, w1_ref, b1_ref, w2_ref, out_ref):
        dinv = dinvb_ref[...]
        u = dinv * (p_ref[0] + p_ref[1] + xs_ref[...])
        h = jnp.dot(u, w1_ref[...], preferred_element_type=jnp.float32)
        h = jnp.maximum(h + b1_ref[...], 0.0)
        t = jnp.dot(h, w2_ref[...], preferred_element_type=jnp.float32)
        out_ref[...] = dinv * t

    return pl.pallas_call(
        body,
        grid=(pl.cdiv(N, _BLK),),
        in_specs=[pl.BlockSpec((NC, _BLK, F), lambda i: (0, i, 0)),
                  pl.BlockSpec((_BLK, F), lambda i: (i, 0)),
                  pl.BlockSpec((_BLK, F), lambda i: (i, 0)),
                  pl.BlockSpec((F, H), lambda i: (0, 0)),
                  pl.BlockSpec((1, H), lambda i: (0, 0)),
                  pl.BlockSpec((H, F), lambda i: (0, 0))],
        out_specs=pl.BlockSpec((_BLK, F), lambda i: (i, 0)),
        out_shape=jax.ShapeDtypeStruct((N, F), jnp.float32),
    )(p, xs, dinvb, W1, b1.reshape(1, H), W2)


def _tc_fin(q, ts, dinvb, b2):
    N, F = ts.shape

    def body(q_ref, ts_ref, dinvb_ref, b2_ref, out_ref):
        out_ref[...] = (dinvb_ref[...] * (q_ref[0] + q_ref[1] + ts_ref[...])
                        + b2_ref[...])

    return pl.pallas_call(
        body,
        grid=(pl.cdiv(N, _BLK),),
        in_specs=[pl.BlockSpec((NC, _BLK, F), lambda i: (0, i, 0)),
                  pl.BlockSpec((_BLK, F), lambda i: (i, 0)),
                  pl.BlockSpec((_BLK, F), lambda i: (i, 0)),
                  pl.BlockSpec((1, F), lambda i: (0, 0))],
        out_specs=pl.BlockSpec((_BLK, F), lambda i: (i, 0)),
        out_shape=jax.ShapeDtypeStruct((N, F), jnp.float32),
    )(q, ts, dinvb, b2.reshape(1, F))


def kernel(x, edge_index, W1, b1, W2, b2):
    N, F = x.shape
    E = edge_index.shape[1]
    R = ((N + 127) // 128) * 128
    EP = pl.cdiv(E, CHUNK * NW * 8) * (CHUNK * NW * 8)
    RW = EP // (CHUNK * NW)
    padn = EP - E

    ei = edge_index.astype(jnp.int32)
    pad_src = (jnp.arange(padn, dtype=jnp.int32) * 97) % N
    pad_dst = N + (jnp.arange(padn, dtype=jnp.int32) % (R - N))
    src2d = jnp.concatenate([ei[0], pad_src]).reshape(EP // CHUNK, CHUNK)
    dst2d = jnp.concatenate([ei[1], pad_dst]).reshape(EP // CHUNK, CHUNK)
    zdeg = jnp.zeros((R,), jnp.float32)
    zacc = jnp.zeros((R, F), jnp.float32)

    degp = _make_deg_kernel(R, RW)(dst2d, zdeg).reshape(NC, R)
    dinvb, xs = _tc_prep(degp, x)
    agg = _make_agg_kernel(R, RW, F)
    p = agg(xs, src2d, dst2d, zacc).reshape(NC, R, F)
    ts = _tc_mid(p, xs, dinvb, W1, b1, W2)
    q = agg(ts, src2d, dst2d, zacc).reshape(NC, R, F)
    return _tc_fin(q, ts, dinvb, b2)


# speedup vs baseline: 32.0725x; 1.0873x over previous
import functools

import jax
import jax.numpy as jnp
from jax import lax
from jax.experimental import pallas as pl
from jax.experimental.pallas import tpu as pltpu
from jax.experimental.pallas import tpu_sc as plsc

NC = 2
NS = 16
NW = NC * NS
LANES = 16
CHUNK = 128
_BLK = 1024


def _sc_mesh():
    return plsc.VectorSubcoreMesh(
        core_axis_name="c", subcore_axis_name="s",
        num_cores=NC, num_subcores=NS)


def _make_deg_kernel(R, RW):
    rpt = R // NS

    nfull, rem = divmod(rpt, CHUNK)
    sizes = [CHUNK] * nfull + ([rem] if rem else [])

    @functools.partial(
        pl.kernel,
        out_type=jax.ShapeDtypeStruct((NC * R,), jnp.float32),
        mesh=_sc_mesh(),
        scratch_types=[
            pltpu.VMEM((RW, CHUNK), jnp.int32),
            pltpu.VMEM((CHUNK,), jnp.float32),
            pltpu.VMEM((CHUNK,), jnp.float32),
            pltpu.VMEM_SHARED((R,), jnp.float32),
        ],
    )
    def deg_kernel(dst_hbm, zdeg_hbm, out_hbm, idx_v, ones_v, stage_v, hist_s):
        cid = lax.axis_index("c")
        sid = lax.axis_index("s")
        wid = cid * NS + sid
        for k in range(CHUNK // LANES):
            ones_v[pl.ds(k * LANES, LANES)] = jnp.ones((LANES,), jnp.float32)
        pltpu.sync_copy(zdeg_hbm.at[pl.ds(0, CHUNK)], stage_v)
        off = 0
        for sz in sizes:
            pltpu.sync_copy(stage_v.at[pl.ds(0, sz)],
                            hist_s.at[pl.ds(sid * rpt + off, sz)])
            off += sz
        pltpu.sync_copy(dst_hbm.at[pl.ds(wid * RW, RW)], idx_v)
        plsc.subcore_barrier()

        def body(j, carry):
            pltpu.sync_copy(ones_v, hist_s.at[idx_v.at[j]], add=True)
            return carry

        lax.fori_loop(0, RW, body, 0)
        plsc.subcore_barrier()
        off = 0
        for sz in sizes:
            pltpu.sync_copy(hist_s.at[pl.ds(sid * rpt + off, sz)],
                            stage_v.at[pl.ds(0, sz)])
            pltpu.sync_copy(stage_v.at[pl.ds(0, sz)],
                            out_hbm.at[pl.ds(cid * R + sid * rpt + off, sz)])
            off += sz

    return deg_kernel


def _make_agg_kernel(R, RW, F):
    rpt = R // NS
    nfull, rem = divmod(rpt, CHUNK)
    sizes = [CHUNK] * nfull + ([rem] if rem else [])
    assert RW % 16 == 0 or RW % 8 == 0
    HALF = RW // 2

    @functools.partial(
        pl.kernel,
        out_type=jax.ShapeDtypeStruct((NC * R, F), jnp.float32),
        mesh=_sc_mesh(),
        scratch_types=[
            pltpu.VMEM((HALF, CHUNK), jnp.int32),
            pltpu.VMEM((HALF, CHUNK), jnp.int32),
            pltpu.VMEM((CHUNK, F), jnp.float32),
            pltpu.VMEM((CHUNK, F), jnp.float32),
            pltpu.VMEM_SHARED((R, F), jnp.float32),
            pltpu.SemaphoreType.DMA,
            pltpu.SemaphoreType.DMA,
            pltpu.SemaphoreType.DMA,
            pltpu.SemaphoreType.DMA,
        ],
    )
    def agg_kernel(xs_hbm, src_hbm, dst_hbm, zacc_hbm, out_hbm,
                   src_v, dst_v, buf0, buf1, acc_s, gsem0, gsem1, ssem0, ssem1):
        cid = lax.axis_index("c")
        sid = lax.axis_index("s")
        wid = cid * NS + sid
        pltpu.sync_copy(zacc_hbm.at[pl.ds(0, CHUNK)], buf0)
        hs = []
        off = 0
        for i, sz in enumerate(sizes):
            hs.append(pltpu.async_copy(
                buf0.at[pl.ds(0, sz)],
                acc_s.at[pl.ds(sid * rpt + off, sz)],
                ssem0 if i % 2 == 0 else ssem1))
            off += sz
        for h in hs:
            h.wait()
        plsc.subcore_barrier()

        def g_wait():
            pltpu.make_async_copy(xs_hbm.at[src_v.at[0]], buf0, gsem0).wait()

        def g_wait1():
            pltpu.make_async_copy(xs_hbm.at[src_v.at[0]], buf1, gsem1).wait()

        def s_wait():
            pltpu.make_async_copy(buf0, acc_s.at[dst_v.at[0]], ssem0).wait()

        def s_wait1():
            pltpu.make_async_copy(buf1, acc_s.at[dst_v.at[0]], ssem1).wait()

        npair = HALF // 2

        def body(jj, carry):
            j0 = 2 * jj
            j1 = j0 + 1
            g_wait()

            @pl.when(jj > 0)
            def _():
                s_wait1()

            pltpu.async_copy(xs_hbm.at[src_v.at[j1]], buf1, gsem1)
            pltpu.async_copy(buf0, acc_s.at[dst_v.at[j0]], ssem0, add=True)
            g_wait1()
            s_wait()

            @pl.when(jj < npair - 1)
            def _():
                pltpu.async_copy(xs_hbm.at[src_v.at[j0 + 2]], buf0, gsem0)

            pltpu.async_copy(buf1, acc_s.at[dst_v.at[j1]], ssem1, add=True)
            return carry

        for half in range(2):
            pltpu.sync_copy(src_hbm.at[pl.ds(wid * RW + half * HALF, HALF)],
                            src_v)
            pltpu.sync_copy(dst_hbm.at[pl.ds(wid * RW + half * HALF, HALF)],
                            dst_v)
            pltpu.async_copy(xs_hbm.at[src_v.at[0]], buf0, gsem0)
            lax.fori_loop(0, npair, body, 0)
            s_wait1()
        plsc.subcore_barrier()
        off = 0
        whs = [None, None]
        for i, sz in enumerate(sizes):
            b = buf0 if i % 2 == 0 else buf1
            sem = ssem0 if i % 2 == 0 else ssem1
            if whs[i % 2] is not None:
                whs[i % 2].wait()
            pltpu.sync_copy(acc_s.at[pl.ds(sid * rpt + off, sz)],
                            b.at[pl.ds(0, sz)])
            whs[i % 2] = pltpu.async_copy(
                b.at[pl.ds(0, sz)],
                out_hbm.at[pl.ds(cid * R + sid * rpt + off, sz), :], sem)
            off += sz
        for h in whs:
            if h is not None:
                h.wait()

    return agg_kernel


def _tc_prep(degp, x):
    N, F = x.shape

    def body(degp_ref, x_ref, dinvb_ref, xs_ref):
        deg = degp_ref[0, :] + degp_ref[1, :] + 1.0
        dinv = lax.rsqrt(deg)
        d2 = jnp.broadcast_to(dinv[:, None], x_ref.shape)
        dinvb_ref[...] = d2
        xs_ref[...] = x_ref[...] * d2

    return pl.pallas_call(
        body,
        grid=(pl.cdiv(N, _BLK),),
        in_specs=[pl.BlockSpec((NC, _BLK), lambda i: (0, i)),
                  pl.BlockSpec((_BLK, F), lambda i: (i, 0))],
        out_specs=[pl.BlockSpec((_BLK, F), lambda i: (i, 0)),
                   pl.BlockSpec((_BLK, F), lambda i: (i, 0))],
        out_shape=[jax.ShapeDtypeStruct((N, F), jnp.float32),
                   jax.ShapeDtypeStruct((N, F), jnp.float32)],
    )(degp, x)


def _tc_mid(p, xs, dinvb, W1, b1, W2):
    N, F = xs.shape
    H = W1.shape[1]

    def body(p_ref, xs_ref, dinvb_ref, w1_ref, b1_ref, w2_ref, out_ref):
        dinv = dinvb_ref[...]
        u = dinv * (p_ref[0] + p_ref[1] + xs_ref[...])
        h = jnp.dot(u, w1_ref[...], preferred_element_type=jnp.float32)
        h = jnp.maximum(h + b1_ref[...], 0.0)
        t = jnp.dot(h, w2_ref[...], preferred_element_type=jnp.float32)
        out_ref[...] = dinv * t

    return pl.pallas_call(
        body,
        grid=(pl.cdiv(N, _BLK),),
        in_specs=[pl.BlockSpec((NC, _BLK, F), lambda i: (0, i, 0)),
                  pl.BlockSpec((_BLK, F), lambda i: (i, 0)),
                  pl.BlockSpec((_BLK, F), lambda i: (i, 0)),
                  pl.BlockSpec((F, H), lambda i: (0, 0)),
                  pl.BlockSpec((1, H), lambda i: (0, 0)),
                  pl.BlockSpec((H, F), lambda i: (0, 0))],
        out_specs=pl.BlockSpec((_BLK, F), lambda i: (i, 0)),
        out_shape=jax.ShapeDtypeStruct((N, F), jnp.float32),
    )(p, xs, dinvb, W1, b1.reshape(1, H), W2)


def _tc_fin(q, ts, dinvb, b2):
    N, F = ts.shape

    def body(q_ref, ts_ref, dinvb_ref, b2_ref, out_ref):
        out_ref[...] = (dinvb_ref[...] * (q_ref[0] + q_ref[1] + ts_ref[...])
                        + b2_ref[...])

    return pl.pallas_call(
        body,
        grid=(pl.cdiv(N, _BLK),),
        in_specs=[pl.BlockSpec((NC, _BLK, F), lambda i: (0, i, 0)),
                  pl.BlockSpec((_BLK, F), lambda i: (i, 0)),
                  pl.BlockSpec((_BLK, F), lambda i: (i, 0)),
                  pl.BlockSpec((1, F), lambda i: (0, 0))],
        out_specs=pl.BlockSpec((_BLK, F), lambda i: (i, 0)),
        out_shape=jax.ShapeDtypeStruct((N, F), jnp.float32),
    )(q, ts, dinvb, b2.reshape(1, F))


def kernel(x, edge_index, W1, b1, W2, b2):
    N, F = x.shape
    E = edge_index.shape[1]
    R = ((N + 127) // 128) * 128
    EP = pl.cdiv(E, CHUNK * NW * 8) * (CHUNK * NW * 8)
    RW = EP // (CHUNK * NW)
    padn = EP - E

    ei = edge_index.astype(jnp.int32)
    pad_src = (jnp.arange(padn, dtype=jnp.int32) * 97) % N
    pad_dst = N + (jnp.arange(padn, dtype=jnp.int32) % (R - N))
    src2d = jnp.concatenate([ei[0], pad_src]).reshape(EP // CHUNK, CHUNK)
    dst2d = jnp.concatenate([ei[1], pad_dst]).reshape(EP // CHUNK, CHUNK)
    zdeg = jnp.zeros((R,), jnp.float32)
    zacc = jnp.zeros((R, F), jnp.float32)

    degp = _make_deg_kernel(R, RW)(dst2d, zdeg).reshape(NC, R)
    dinvb, xs = _tc_prep(degp, x)
    agg = _make_agg_kernel(R, RW, F)
    p = agg(xs, src2d, dst2d, zacc).reshape(NC, R, F)
    ts = _tc_mid(p, xs, dinvb, W1, b1, W2)
    q = agg(ts, src2d, dst2d, zacc).reshape(NC, R, F)
    return _tc_fin(q, ts, dinvb, b2)


# speedup vs baseline: 32.5235x; 1.0141x over previous
import functools

import jax
import jax.numpy as jnp
from jax import lax
from jax.experimental import pallas as pl
from jax.experimental.pallas import tpu as pltpu
from jax.experimental.pallas import tpu_sc as plsc

NC = 2
NS = 16
NW = NC * NS
LANES = 16
CHUNK = 128
_BLK = 1024


def _sc_mesh():
    return plsc.VectorSubcoreMesh(
        core_axis_name="c", subcore_axis_name="s",
        num_cores=NC, num_subcores=NS)


def _make_deg_kernel(R, RW):
    rpt = R // NS

    nfull, rem = divmod(rpt, CHUNK)
    sizes = [CHUNK] * nfull + ([rem] if rem else [])

    @functools.partial(
        pl.kernel,
        out_type=jax.ShapeDtypeStruct((NC * R,), jnp.float32),
        mesh=_sc_mesh(),
        scratch_types=[
            pltpu.VMEM((RW, CHUNK), jnp.int32),
            pltpu.VMEM((CHUNK,), jnp.float32),
            pltpu.VMEM((CHUNK,), jnp.float32),
            pltpu.VMEM_SHARED((R,), jnp.float32),
            pltpu.SemaphoreType.DMA,
            pltpu.SemaphoreType.DMA,
        ],
    )
    def deg_kernel(dst_hbm, zdeg_hbm, out_hbm, idx_v, ones_v, stage_v, hist_s,
                   dsem0, dsem1):
        cid = lax.axis_index("c")
        sid = lax.axis_index("s")
        wid = cid * NS + sid
        for k in range(CHUNK // LANES):
            ones_v[pl.ds(k * LANES, LANES)] = jnp.ones((LANES,), jnp.float32)
        pltpu.sync_copy(zdeg_hbm.at[pl.ds(0, CHUNK)], stage_v)
        off = 0
        for sz in sizes:
            pltpu.sync_copy(stage_v.at[pl.ds(0, sz)],
                            hist_s.at[pl.ds(sid * rpt + off, sz)])
            off += sz
        pltpu.sync_copy(dst_hbm.at[pl.ds(wid * RW, RW)], idx_v)
        plsc.subcore_barrier()

        def body(jj, carry):
            hs = [pltpu.async_copy(ones_v, hist_s.at[idx_v.at[8 * jj + k]],
                                   dsem0 if k % 2 == 0 else dsem1, add=True)
                  for k in range(8)]
            for h in hs:
                h.wait()
            return carry

        lax.fori_loop(0, RW // 8, body, 0)
        plsc.subcore_barrier()
        off = 0
        for sz in sizes:
            pltpu.sync_copy(hist_s.at[pl.ds(sid * rpt + off, sz)],
                            stage_v.at[pl.ds(0, sz)])
            pltpu.sync_copy(stage_v.at[pl.ds(0, sz)],
                            out_hbm.at[pl.ds(cid * R + sid * rpt + off, sz)])
            off += sz

    return deg_kernel


def _make_agg_kernel(R, RW, F):
    rpt = R // NS
    nfull, rem = divmod(rpt, CHUNK)
    sizes = [CHUNK] * nfull + ([rem] if rem else [])
    assert RW % 16 == 0 or RW % 8 == 0
    HALF = RW // 2

    @functools.partial(
        pl.kernel,
        out_type=jax.ShapeDtypeStruct((NC * R, F), jnp.float32),
        mesh=_sc_mesh(),
        scratch_types=[
            pltpu.VMEM((HALF, CHUNK), jnp.int32),
            pltpu.VMEM((HALF, CHUNK), jnp.int32),
            pltpu.VMEM((CHUNK, F), jnp.float32),
            pltpu.VMEM((CHUNK, F), jnp.float32),
            pltpu.VMEM_SHARED((R, F), jnp.float32),
            pltpu.SemaphoreType.DMA,
            pltpu.SemaphoreType.DMA,
            pltpu.SemaphoreType.DMA,
            pltpu.SemaphoreType.DMA,
        ],
    )
    def agg_kernel(xs_hbm, src_hbm, dst_hbm, zacc_hbm, out_hbm,
                   src_v, dst_v, buf0, buf1, acc_s, gsem0, gsem1, ssem0, ssem1):
        cid = lax.axis_index("c")
        sid = lax.axis_index("s")
        wid = cid * NS + sid
        pltpu.sync_copy(zacc_hbm.at[pl.ds(0, CHUNK)], buf0)
        hs = []
        off = 0
        for i, sz in enumerate(sizes):
            hs.append(pltpu.async_copy(
                buf0.at[pl.ds(0, sz)],
                acc_s.at[pl.ds(sid * rpt + off, sz)],
                ssem0 if i % 2 == 0 else ssem1))
            off += sz
        for h in hs:
            h.wait()
        plsc.subcore_barrier()

        def g_wait():
            pltpu.make_async_copy(xs_hbm.at[src_v.at[0]], buf0, gsem0).wait()

        def g_wait1():
            pltpu.make_async_copy(xs_hbm.at[src_v.at[0]], buf1, gsem1).wait()

        def s_wait():
            pltpu.make_async_copy(buf0, acc_s.at[dst_v.at[0]], ssem0).wait()

        def s_wait1():
            pltpu.make_async_copy(buf1, acc_s.at[dst_v.at[0]], ssem1).wait()

        npair = HALF // 2

        def body(jj, carry):
            j0 = 2 * jj
            j1 = j0 + 1
            g_wait()

            @pl.when(jj > 0)
            def _():
                s_wait1()

            pltpu.async_copy(xs_hbm.at[src_v.at[j1]], buf1, gsem1)
            pltpu.async_copy(buf0, acc_s.at[dst_v.at[j0]], ssem0, add=True)
            g_wait1()
            s_wait()

            @pl.when(jj < npair - 1)
            def _():
                pltpu.async_copy(xs_hbm.at[src_v.at[j0 + 2]], buf0, gsem0)

            pltpu.async_copy(buf1, acc_s.at[dst_v.at[j1]], ssem1, add=True)
            return carry

        for half in range(2):
            pltpu.sync_copy(src_hbm.at[pl.ds(wid * RW + half * HALF, HALF)],
                            src_v)
            pltpu.sync_copy(dst_hbm.at[pl.ds(wid * RW + half * HALF, HALF)],
                            dst_v)
            pltpu.async_copy(xs_hbm.at[src_v.at[0]], buf0, gsem0)
            lax.fori_loop(0, npair, body, 0)
            s_wait1()
        plsc.subcore_barrier()
        off = 0
        whs = [None, None]
        for i, sz in enumerate(sizes):
            b = buf0 if i % 2 == 0 else buf1
            sem = ssem0 if i % 2 == 0 else ssem1
            if whs[i % 2] is not None:
                whs[i % 2].wait()
            pltpu.sync_copy(acc_s.at[pl.ds(sid * rpt + off, sz)],
                            b.at[pl.ds(0, sz)])
            whs[i % 2] = pltpu.async_copy(
                b.at[pl.ds(0, sz)],
                out_hbm.at[pl.ds(cid * R + sid * rpt + off, sz), :], sem)
            off += sz
        for h in whs:
            if h is not None:
                h.wait()

    return agg_kernel


def _tc_prep(degp, x):
    N, F = x.shape

    def body(degp_ref, x_ref, dinvb_ref, xs_ref):
        deg = degp_ref[0, :] + degp_ref[1, :] + 1.0
        dinv = lax.rsqrt(deg)
        d2 = jnp.broadcast_to(dinv[:, None], x_ref.shape)
        dinvb_ref[...] = d2
        xs_ref[...] = x_ref[...] * d2

    return pl.pallas_call(
        body,
        grid=(pl.cdiv(N, _BLK),),
        in_specs=[pl.BlockSpec((NC, _BLK), lambda i: (0, i)),
                  pl.BlockSpec((_BLK, F), lambda i: (i, 0))],
        out_specs=[pl.BlockSpec((_BLK, F), lambda i: (i, 0)),
                   pl.BlockSpec((_BLK, F), lambda i: (i, 0))],
        out_shape=[jax.ShapeDtypeStruct((N, F), jnp.float32),
                   jax.ShapeDtypeStruct((N, F), jnp.float32)],
    )(degp, x)


def _tc_mid(p, xs, dinvb, W1, b1, W2):
    N, F = xs.shape
    H = W1.shape[1]

    def body(p_ref, xs_ref, dinvb_ref, w1_ref, b1_ref, w2_ref, out_ref):
        dinv = dinvb_ref[...]
        u = dinv * (p_ref[0] + p_ref[1] + xs_ref[...])
        h = jnp.dot(u, w1_ref[...], preferred_element_type=jnp.float32)
        h = jnp.maximum(h + b1_ref[...], 0.0)
        t = jnp.dot(h, w2_ref[...], preferred_element_type=jnp.float32)
        out_ref[...] = dinv * t

    return pl.pallas_call(
        body,
        grid=(pl.cdiv(N, _BLK),),
        in_specs=[pl.BlockSpec((NC, _BLK, F), lambda i: (0, i, 0)),
                  pl.BlockSpec((_BLK, F), lambda i: (i, 0)),
                  pl.BlockSpec((_BLK, F), lambda i: (i, 0)),
                  pl.BlockSpec((F, H), lambda i: (0, 0)),
                  pl.BlockSpec((1, H), lambda i: (0, 0)),
                  pl.BlockSpec((H, F), lambda i: (0, 0))],
        out_specs=pl.BlockSpec((_BLK, F), lambda i: (i, 0)),
        out_shape=jax.ShapeDtypeStruct((N, F), jnp.float32),
    )(p, xs, dinvb, W1, b1.reshape(1, H), W2)


def _tc_fin(q, ts, dinvb, b2):
    N, F = ts.shape

    def body(q_ref, ts_ref, dinvb_ref, b2_ref, out_ref):
        out_ref[...] = (dinvb_ref[...] * (q_ref[0] + q_ref[1] + ts_ref[...])
                        + b2_ref[...])

    return pl.pallas_call(
        body,
        grid=(pl.cdiv(N, _BLK),),
        in_specs=[pl.BlockSpec((NC, _BLK, F), lambda i: (0, i, 0)),
                  pl.BlockSpec((_BLK, F), lambda i: (i, 0)),
                  pl.BlockSpec((_BLK, F), lambda i: (i, 0)),
                  pl.BlockSpec((1, F), lambda i: (0, 0))],
        out_specs=pl.BlockSpec((_BLK, F), lambda i: (i, 0)),
        out_shape=jax.ShapeDtypeStruct((N, F), jnp.float32),
    )(q, ts, dinvb, b2.reshape(1, F))


def kernel(x, edge_index, W1, b1, W2, b2):
    N, F = x.shape
    E = edge_index.shape[1]
    R = ((N + 127) // 128) * 128
    EP = pl.cdiv(E, CHUNK * NW * 8) * (CHUNK * NW * 8)
    RW = EP // (CHUNK * NW)
    padn = EP - E

    ei = edge_index.astype(jnp.int32)
    pad_src = (jnp.arange(padn, dtype=jnp.int32) * 97) % N
    pad_dst = N + (jnp.arange(padn, dtype=jnp.int32) % (R - N))
    src2d = jnp.concatenate([ei[0], pad_src]).reshape(EP // CHUNK, CHUNK)
    dst2d = jnp.concatenate([ei[1], pad_dst]).reshape(EP // CHUNK, CHUNK)
    zdeg = jnp.zeros((R,), jnp.float32)
    zacc = jnp.zeros((R, F), jnp.float32)

    degp = _make_deg_kernel(R, RW)(dst2d, zdeg).reshape(NC, R)
    dinvb, xs = _tc_prep(degp, x)
    agg = _make_agg_kernel(R, RW, F)
    p = agg(xs, src2d, dst2d, zacc).reshape(NC, R, F)
    ts = _tc_mid(p, xs, dinvb, W1, b1, W2)
    q = agg(ts, src2d, dst2d, zacc).reshape(NC, R, F)
    return _tc_fin(q, ts, dinvb, b2)
